# Initial kernel scaffold; baseline (speedup 1.0000x reference)
import jax
import jax.numpy as jnp
from jax.experimental import pallas as pl


def kernel(params, solute_x, solute_edge_index, solute_edge_attr, solute_global_features, solute_batch, solvent_x, solvent_edge_index, solvent_edge_attr, solvent_global_features, solvent_batch):
    raise NotImplementedError("write your pallas kernel here")


# speedup vs baseline: 1.1392x; 1.1392x over previous
import jax
import jax.numpy as jnp
from jax.experimental import pallas as pl

N = 10000
E = 160000
NG = 256
H = 8


def _bn(x, g, b):
    return x * (g / jnp.sqrt(1.0 + 1e-05)) + b


def _gat(x, src, dst, W, asrc, adst, b, c):
    n = x.shape[0]
    xp = (x @ W).reshape(n, H, c)
    a_s = jnp.einsum('nhc,hc->nh', xp, asrc)
    a_d = jnp.einsum('nhc,hc->nh', xp, adst)
    e = jax.nn.leaky_relu(a_s[src] + a_d[dst], 0.2)
    ee = jnp.exp(e)
    ee_loop = jnp.exp(jax.nn.leaky_relu(a_s + a_d, 0.2))
    den = jax.ops.segment_sum(ee, dst, num_segments=n) + ee_loop
    agg = jax.ops.segment_sum(xp[src] * ee[:, :, None], dst, num_segments=n)
    agg = agg + xp * ee_loop[:, :, None]
    out = agg / den[:, :, None]
    return out.reshape(n, H * c) + b


def _branch(p, g, x, ei, gfeat, batch):
    src = ei[0]
    dst = ei[1]
    h = x
    for i in (1, 2, 3):
        pre = '%s_conv%d' % (g, i)
        hn = jax.nn.relu(_gat(h, src, dst, p[pre + '_W'], p[pre + '_asrc'], p[pre + '_adst'], p[pre + '_b'], 64))
        hn = _bn(hn, p['%s_bn%d_g' % (g, i)], p['%s_bn%d_b' % (g, i)])
        rp = '%s_res%d' % (g, i)
        hn = hn + _bn(h @ p[rp + '_W'] + p[rp + '_b'], p[rp + '_g'], p[rp + '_bb'])
        h = hn
    pre = g + '_conv4'
    h4 = jax.nn.relu(_gat(h, src, dst, p[pre + '_W'], p[pre + '_asrc'], p[pre + '_adst'], p[pre + '_b'], 32))
    h4 = _bn(h4, p[g + '_bn4_g'], p[g + '_bn4_b'])
    onehot = (batch[None, :] == jnp.arange(NG, dtype=batch.dtype)[:, None]).astype(h4.dtype)
    pooled = onehot @ h4
    return jnp.concatenate([pooled, gfeat], axis=1)


def _fc_head_kernel(x_ref, *refs):
    (w1, b1, w2, b2, w3, b3, w4, b4,
     g1, bb1, g2, bb2, g3, bb3,
     rw1, rb1, rg1, rbb1, rw2, rb2, rg2, rbb2, rw3, rb3, rg3, rbb3,
     out_ref) = refs
    x = x_ref[...]
    ws = (w1, w2, w3)
    bs = (b1, b2, b3)
    gs = (g1, g2, g3)
    bbs = (bb1, bb2, bb3)
    rws = (rw1, rw2, rw3)
    rbs = (rb1, rb2, rb3)
    rgs = (rg1, rg2, rg3)
    rbbs = (rbb1, rbb2, rbb3)
    inv = 1.0 / jnp.sqrt(1.0 + 1e-05)
    for i in range(3):
        r = (x @ rws[i][...] + rbs[i][...]) * (rgs[i][...] * inv) + rbbs[i][...]
        y = jnp.maximum(x @ ws[i][...] + bs[i][...], 0.0)
        x = y * (gs[i][...] * inv) + bbs[i][...] + r
    out_ref[...] = x @ w4[...] + b4[...]


def _fc_head(p, x):
    args = [x, p['fc1_W'], p['fc1_b'], p['fc2_W'], p['fc2_b'], p['fc3_W'], p['fc3_b'], p['fc4_W'], p['fc4_b']]
    for i in (1, 2, 3):
        args += [p['fcbn%d_g' % i], p['fcbn%d_b' % i]]
    for i in (1, 2, 3):
        args += [p['fcres%d_W' % i], p['fcres%d_b' % i], p['fcres%d_g' % i], p['fcres%d_bb' % i]]
    return pl.pallas_call(
        _fc_head_kernel,
        out_shape=jax.ShapeDtypeStruct((x.shape[0], 1), jnp.float32),
    )(*args)


def kernel(params, solute_x, solute_edge_index, solute_edge_attr, solute_global_features, solute_batch, solvent_x, solvent_edge_index, solvent_edge_attr, solvent_global_features, solvent_batch):
    del solute_edge_attr, solvent_edge_attr
    p = params
    x1 = _branch(p, 's1', solute_x, solute_edge_index, solute_global_features, solute_batch)
    x2 = _branch(p, 's2', solvent_x, solvent_edge_index, solvent_global_features, solvent_batch)
    x = jnp.concatenate([x1, x2], axis=1)
    return _fc_head(p, x)


# speedup vs baseline: 16.5929x; 14.5652x over previous
import functools

import jax
import jax.numpy as jnp
from jax import lax
from jax.experimental import pallas as pl
from jax.experimental.pallas import tpu as pltpu
from jax.experimental.pallas import tpu_sc as plsc

N = 10000
E = 160000
NG = 256
H = 8

NW = 32
KB = 128
EPW = 5120
EP = NW * EPW
NBLK = EPW // KB
NP = 10240
NPT = NP // 16

_BN_INV = 1.0 / (1.0 + 1e-05) ** 0.5


def _edge_body(c, *refs):
    (src2, dst2, asd, *rest) = refs
    xps = rest[0:8]
    den2 = rest[8]
    aggs = rest[9:17]
    s_idx, d_idx, rows_s, rows_d = rest[17:21]
    eetts = rest[21:29]
    stage0, stage1, eepad, den_sp, agg_sp, sem0, sem1 = rest[29:]

    core = lax.axis_index("c")
    tile = lax.axis_index("s")
    w = tile * 2 + core
    rowbase = w * NBLK
    iota16 = lax.iota(jnp.int32, 16)
    zeros16 = jnp.zeros((16,), jnp.float32)

    pltpu.sync_copy(src2.at[pl.ds(rowbase, NBLK)], s_idx)
    pltpu.sync_copy(dst2.at[pl.ds(rowbase, NBLK)], d_idx)
    for r in range(KB):
        eepad[r, :] = zeros16
    for i in range(NPT // 128):
        pltpu.sync_copy(eepad, den_sp.at[pl.ds(tile * NPT + i * 128, 128)])
    plsc.subcore_barrier()

    def blk_a(blk, carry):
        pltpu.sync_copy(asd.at[s_idx.at[blk]], rows_s)
        pltpu.sync_copy(asd.at[d_idx.at[blk]], rows_d)
        for g in range(8):
            gi = g * 16 + iota16
            gpos = w * EPW + blk * KB + g * 16 + iota16
            valid = gpos < E
            for h in range(8):
                hvec = jnp.full((16,), h, jnp.int32)
                a_s = plsc.load_gather(rows_s, [gi, hvec])
                a_d = plsc.load_gather(rows_d, [gi, jnp.full((16,), 8 + h, jnp.int32)])
                v = a_s + a_d
                ee = jnp.exp(jnp.where(v >= 0.0, v, 0.2 * v))
                ee = jnp.where(valid, ee, 0.0)
                eetts[h][pl.ds(blk * KB + g * 16, 16)] = ee
                plsc.store_scatter(eepad, [gi, hvec], ee)
        pltpu.sync_copy(eepad, den_sp.at[d_idx.at[blk]], add=True)
        return carry

    lax.fori_loop(0, NBLK, blk_a, 0)
    plsc.subcore_barrier()
    pltpu.sync_copy(den_sp.at[pl.ds(tile * NPT, NPT)],
                    den2.at[core].at[pl.ds(tile * NPT, NPT)])

    stages = (stage0, stage1)
    sems = (sem0, sem1)
    for h in range(8):
        plsc.subcore_barrier()

        def zrow(r, carry):
            rvec = jnp.full((16,), r, jnp.int32)
            for cc in range(c // 16):
                plsc.store_scatter(stage0, [rvec, cc * 16 + iota16], zeros16)
            return carry

        lax.fori_loop(0, KB, zrow, 0)
        for i in range(NPT // 128):
            pltpu.sync_copy(stage0, agg_sp.at[pl.ds(tile * NPT + i * 128, 128)])
        plsc.subcore_barrier()

        pltpu.async_copy(xps[h].at[s_idx.at[0]], stage0, sem0)

        def pair_b(p, carry):
            for b in range(2):
                blk = 2 * p + b
                stg = stages[b]
                pltpu.make_async_copy(xps[h].at[s_idx.at[blk]], stg, sems[b]).wait()

                @pl.when(blk + 1 < NBLK)
                def _():
                    pltpu.async_copy(xps[h].at[s_idx.at[blk + 1]],
                                     stages[1 - b], sems[1 - b])

                def scale16(j16, carry2):
                    for jj in range(8):
                        j = j16 * 8 + jj
                        jvec = jnp.full((16,), j, jnp.int32)
                        eev = plsc.load_gather(
                            eetts[h], [jnp.full((16,), blk * KB, jnp.int32) + jvec])
                        for cc in range(c // 16):
                            ci = cc * 16 + iota16
                            v = plsc.load_gather(stg, [jvec, ci])
                            plsc.store_scatter(stg, [jvec, ci], v * eev)
                    return carry2

                lax.fori_loop(0, KB // 8, scale16, 0)
                pltpu.sync_copy(stg, agg_sp.at[d_idx.at[blk]], add=True)
            return carry

        lax.fori_loop(0, NBLK // 2, pair_b, 0)
        plsc.subcore_barrier()
        pltpu.sync_copy(agg_sp.at[pl.ds(tile * NPT, NPT)],
                        aggs[h].at[core].at[pl.ds(tile * NPT, NPT)])


def _gat_edge_sc(src2, dst2, asd, xps, c):
    mesh = plsc.VectorSubcoreMesh(core_axis_name="c", subcore_axis_name="s")
    out_type = ([jax.ShapeDtypeStruct((2, NP, 16), jnp.float32)] +
                [jax.ShapeDtypeStruct((2, NP, c), jnp.float32) for _ in range(8)])
    scratch = [
        pltpu.VMEM((NBLK, KB), jnp.int32),
        pltpu.VMEM((NBLK, KB), jnp.int32),
        pltpu.VMEM((KB, 16), jnp.float32),
        pltpu.VMEM((KB, 16), jnp.float32),
    ] + [pltpu.VMEM((EPW,), jnp.float32) for _ in range(8)] + [
        pltpu.VMEM((KB, c), jnp.float32),
        pltpu.VMEM((KB, c), jnp.float32),
        pltpu.VMEM((KB, 16), jnp.float32),
        pltpu.VMEM_SHARED((NP, 16), jnp.float32),
        pltpu.VMEM_SHARED((NP, c), jnp.float32),
        pltpu.SemaphoreType.DMA,
        pltpu.SemaphoreType.DMA,
    ]
    outs = pl.kernel(
        functools.partial(_edge_body, c),
        out_type=out_type,
        mesh=mesh,
        scratch_types=scratch,
        compiler_params=pltpu.CompilerParams(needs_layout_passes=False,
                                             use_tc_tiling_on_sc=False),
    )(src2, dst2, asd, *xps)
    return outs[0], outs[1:]


def _mm_body(x_ref, w_ref, o_ref):
    o_ref[...] = jnp.dot(x_ref[...], w_ref[...],
                         preferred_element_type=jnp.float32)


def _matmul(x, w, bm=400):
    m, k = x.shape
    n = w.shape[1]
    return pl.pallas_call(
        _mm_body,
        grid=(m // bm,),
        in_specs=[pl.BlockSpec((bm, k), lambda i: (i, 0)),
                  pl.BlockSpec((k, n), lambda i: (0, 0))],
        out_specs=pl.BlockSpec((bm, n), lambda i: (i, 0)),
        out_shape=jax.ShapeDtypeStruct((m, n), jnp.float32),
    )(x, w)


def _epi_body(c, with_res, *refs):
    if with_res:
        (den_ref, asd_ref, rpre_ref, b_ref, g_ref, bb_ref,
         rb_ref, rg_ref, rbb_ref, *rest) = refs
    else:
        (den_ref, asd_ref, b_ref, g_ref, bb_ref, *rest) = refs
    aggs = rest[0:8]
    xps = rest[8:16]
    o_ref = rest[16]
    cols = []
    for h in range(8):
        aggr = aggs[h][0] + aggs[h][1]
        xp = xps[h][...]
        d = den_ref[0, :, h:h + 1] + den_ref[1, :, h:h + 1]
        s = asd_ref[:, h:h + 1] + asd_ref[:, 8 + h:9 + h]
        eel = jnp.exp(jnp.where(s >= 0.0, s, 0.2 * s))
        cols.append((aggr + xp * eel) / (d + eel))
    x = jnp.concatenate(cols, axis=1) + b_ref[...]
    x = jnp.maximum(x, 0.0)
    x = x * (g_ref[...] * _BN_INV) + bb_ref[...]
    if with_res:
        r = (rpre_ref[...] + rb_ref[...]) * (rg_ref[...] * _BN_INV) + rbb_ref[...]
        x = x + r
    o_ref[...] = x


def _epilogue(den2, asd, aggs, xps, b, g, bb, c, rpre=None, rb=None, rg=None,
              rbb=None, bm=400):
    with_res = rpre is not None
    hc = H * c
    vec = lambda v: v.reshape(1, -1)
    args = [den2, asd]
    in_specs = [pl.BlockSpec((2, bm, 16), lambda i: (0, i, 0)),
                pl.BlockSpec((bm, 16), lambda i: (i, 0))]
    if with_res:
        args += [rpre]
        in_specs += [pl.BlockSpec((bm, hc), lambda i: (i, 0))]
    args += [vec(b), vec(g), vec(bb)]
    in_specs += [pl.BlockSpec((1, hc), lambda i: (0, 0))] * 3
    if with_res:
        args += [vec(rb), vec(rg), vec(rbb)]
        in_specs += [pl.BlockSpec((1, hc), lambda i: (0, 0))] * 3
    args += list(aggs) + list(xps)
    in_specs += [pl.BlockSpec((2, bm, c), lambda i: (0, i, 0))] * 8
    in_specs += [pl.BlockSpec((bm, c), lambda i: (i, 0))] * 8
    return pl.pallas_call(
        functools.partial(_epi_body, c, with_res),
        grid=(N // bm,),
        in_specs=in_specs,
        out_specs=pl.BlockSpec((bm, hc), lambda i: (i, 0)),
        out_shape=jax.ShapeDtypeStruct((N, hc), jnp.float32),
    )(*args)


def _pool_body(b_ref, h_ref, o_ref):
    @pl.when(pl.program_id(0) == 0)
    def _():
        o_ref[...] = jnp.zeros_like(o_ref)

    bi = b_ref[...]
    oh = (bi.T == lax.broadcasted_iota(jnp.int32, (NG, bi.shape[0]), 0))
    o_ref[...] += jnp.dot(oh.astype(jnp.float32), h_ref[...],
                          preferred_element_type=jnp.float32)


def _pool(batch, h4, bm=400):
    return pl.pallas_call(
        _pool_body,
        grid=(N // bm,),
        in_specs=[pl.BlockSpec((bm, 1), lambda i: (i, 0)),
                  pl.BlockSpec((bm, h4.shape[1]), lambda i: (i, 0))],
        out_specs=pl.BlockSpec((NG, h4.shape[1]), lambda i: (0, 0)),
        out_shape=jax.ShapeDtypeStruct((NG, h4.shape[1]), jnp.float32),
    )(batch.reshape(N, 1), h4)


def _fc_head_kernel(x_ref, *refs):
    (w1, b1, w2, b2, w3, b3, w4, b4,
     g1, bb1, g2, bb2, g3, bb3,
     rw1, rb1, rg1, rbb1, rw2, rb2, rg2, rbb2, rw3, rb3, rg3, rbb3,
     out_ref) = refs
    x = x_ref[...]
    ws = (w1, w2, w3)
    bs = (b1, b2, b3)
    gs = (g1, g2, g3)
    bbs = (bb1, bb2, bb3)
    rws = (rw1, rw2, rw3)
    rbs = (rb1, rb2, rb3)
    rgs = (rg1, rg2, rg3)
    rbbs = (rbb1, rbb2, rbb3)
    for i in range(3):
        r = (x @ rws[i][...] + rbs[i][...]) * (rgs[i][...] * _BN_INV) + rbbs[i][...]
        y = jnp.maximum(x @ ws[i][...] + bs[i][...], 0.0)
        x = y * (gs[i][...] * _BN_INV) + bbs[i][...] + r
    out_ref[...] = x @ w4[...] + b4[...]


def _fc_head(p, x):
    args = [x, p['fc1_W'], p['fc1_b'], p['fc2_W'], p['fc2_b'], p['fc3_W'],
            p['fc3_b'], p['fc4_W'], p['fc4_b']]
    for i in (1, 2, 3):
        args += [p['fcbn%d_g' % i], p['fcbn%d_b' % i]]
    for i in (1, 2, 3):
        args += [p['fcres%d_W' % i], p['fcres%d_b' % i], p['fcres%d_g' % i],
                 p['fcres%d_bb' % i]]
    return pl.pallas_call(
        _fc_head_kernel,
        out_shape=jax.ShapeDtypeStruct((x.shape[0], 1), jnp.float32),
    )(*args)


def _head_mixers(asrc, adst, c):
    eye = jnp.eye(H, dtype=jnp.float32)
    a_s = (asrc[:, :, None] * eye[:, None, :]).reshape(H * c, H)
    a_d = (adst[:, :, None] * eye[:, None, :]).reshape(H * c, H)
    return jnp.concatenate([a_s, a_d], axis=1)


def _gat_layer(p, pre, x, src2, dst2, c, res_pre=None):
    W = p[pre + '_W']
    acat = _head_mixers(p[pre + '_asrc'], p[pre + '_adst'], c)
    parts = [W, W @ acat]
    if res_pre is not None:
        parts.append(p[res_pre + '_W'])
    wcat = jnp.concatenate(parts, axis=1)
    xpad = jnp.concatenate(
        [x, jnp.zeros((NP - N, x.shape[1]), jnp.float32)], axis=0)
    out = _matmul(xpad, wcat, bm=512)
    hc = H * c
    asd = out[:, hc:hc + 16]
    xps = [out[:, h * c:(h + 1) * c] for h in range(H)]
    rpre = out[:, hc + 16:] if res_pre is not None else None
    den2, aggs = _gat_edge_sc(src2, dst2, asd, xps, c)
    return den2, asd, aggs, xps, rpre


def _branch(p, g, x, ei, gfeat, batch):
    pad = EP - E
    src2 = jnp.concatenate([ei[0], jnp.zeros((pad,), ei.dtype)]).reshape(EP // KB, KB)
    dst2 = jnp.concatenate([ei[1], jnp.zeros((pad,), ei.dtype)]).reshape(EP // KB, KB)
    h = x
    for i in (1, 2, 3):
        pre = '%s_conv%d' % (g, i)
        rp = '%s_res%d' % (g, i)
        den2, asd, aggs, xps, rpre = _gat_layer(p, pre, h, src2, dst2, 64, rp)
        h = _epilogue(den2, asd, aggs, xps,
                      p[pre + '_b'], p['%s_bn%d_g' % (g, i)],
                      p['%s_bn%d_b' % (g, i)], 64,
                      rpre=rpre, rb=p[rp + '_b'], rg=p[rp + '_g'],
                      rbb=p[rp + '_bb'])
    pre = g + '_conv4'
    den2, asd, aggs, xps, _ = _gat_layer(p, pre, h, src2, dst2, 32)
    h4 = _epilogue(den2, asd, aggs, xps, p[pre + '_b'], p[g + '_bn4_g'],
                   p[g + '_bn4_b'], 32)
    pooled = _pool(batch, h4)
    return jnp.concatenate([pooled, gfeat], axis=1)


def kernel(params, solute_x, solute_edge_index, solute_edge_attr,
           solute_global_features, solute_batch, solvent_x, solvent_edge_index,
           solvent_edge_attr, solvent_global_features, solvent_batch):
    del solute_edge_attr, solvent_edge_attr
    p = params
    x1 = _branch(p, 's1', solute_x, solute_edge_index, solute_global_features,
                 solute_batch)
    x2 = _branch(p, 's2', solvent_x, solvent_edge_index,
                 solvent_global_features, solvent_batch)
    x = jnp.concatenate([x1, x2], axis=1)
    return _fc_head(p, x)


# speedup vs baseline: 16.8049x; 1.0128x over previous
import functools

import jax
import jax.numpy as jnp
from jax import lax
from jax.experimental import pallas as pl
from jax.experimental.pallas import tpu as pltpu
from jax.experimental.pallas import tpu_sc as plsc

N = 10000
E = 160000
NG = 256
H = 8

NW = 32
KB = 128
EPW = 5120
EP = NW * EPW
NBLK = EPW // KB
NP = 10240
NPT = NP // 16

_BN_INV = 1.0 / (1.0 + 1e-05) ** 0.5


def _exp_sc(x):
    kf = x * 1.4426950408889634 + jnp.where(x >= 0.0, 0.5, -0.5)
    k = kf.astype(jnp.int32)
    kf = k.astype(jnp.float32)
    r = x - kf * 0.693359375
    r = r + kf * 2.12194440054690583e-4
    p = jnp.float32(1.0 / 120)
    p = p * r + jnp.float32(1.0 / 24)
    p = p * r + jnp.float32(1.0 / 6)
    p = p * r + jnp.float32(0.5)
    p = p * r + jnp.float32(1.0)
    p = p * r + jnp.float32(1.0)
    scale = plsc.bitcast((k + 127) << 23, jnp.float32)
    return p * scale


def _edge_body(c, *refs):
    (src2, dst2, asd, *rest) = refs
    xps = rest[0:8]
    den2 = rest[8]
    aggs = rest[9:17]
    s_idx, d_idx, rows_s, rows_d = rest[17:21]
    eetts = rest[21:29]
    (stage0, stage1, rows_s1, rows_d1, eepad, den_sp, agg_sp,
     sem0, sem1, ssem0, ssem1) = rest[29:]
    ssems = (ssem0, ssem1)

    core = lax.axis_index("c")
    tile = lax.axis_index("s")
    w = tile * 2 + core
    rowbase = w * NBLK
    iota16 = lax.iota(jnp.int32, 16)
    zeros16 = jnp.zeros((16,), jnp.float32)

    pltpu.sync_copy(src2.at[pl.ds(rowbase, NBLK)], s_idx)
    pltpu.sync_copy(dst2.at[pl.ds(rowbase, NBLK)], d_idx)
    for r in range(KB):
        eepad[r, :] = zeros16
    for i in range(NPT // 128):
        pltpu.sync_copy(eepad, den_sp.at[pl.ds(tile * NPT + i * 128, 128)])
    plsc.subcore_barrier()

    rows_ss = (rows_s, rows_s1)
    rows_ds = (rows_d, rows_d1)
    gsems = (sem0, sem1)
    pltpu.async_copy(asd.at[s_idx.at[0]], rows_s, sem0)
    pltpu.async_copy(asd.at[d_idx.at[0]], rows_d, sem0)

    def pair_a(p, carry):
        for b in range(2):
            blk = 2 * p + b
            rs = rows_ss[b]
            rd = rows_ds[b]
            pltpu.make_async_copy(asd.at[s_idx.at[blk]], rs, gsems[b]).wait()
            pltpu.make_async_copy(asd.at[d_idx.at[blk]], rd, gsems[b]).wait()

            @pl.when(blk + 1 < NBLK)
            def _():
                pltpu.async_copy(asd.at[s_idx.at[blk + 1]],
                                 rows_ss[1 - b], gsems[1 - b])
                pltpu.async_copy(asd.at[d_idx.at[blk + 1]],
                                 rows_ds[1 - b], gsems[1 - b])

            for g in range(8):
                gi = g * 16 + iota16
                gpos = w * EPW + blk * KB + g * 16 + iota16
                valid = gpos < E
                for h in range(8):
                    hvec = jnp.full((16,), h, jnp.int32)
                    a_s = plsc.load_gather(rs, [gi, hvec])
                    a_d = plsc.load_gather(rd, [gi, jnp.full((16,), 8 + h, jnp.int32)])
                    v = a_s + a_d
                    ee = _exp_sc(jnp.where(v >= 0.0, v, 0.2 * v))
                    ee = jnp.where(valid, ee, 0.0)
                    eetts[h][pl.ds(blk * KB + g * 16, 16)] = ee
                    plsc.store_scatter(eepad, [gi, hvec], ee)
            pltpu.sync_copy(eepad, den_sp.at[d_idx.at[blk]], add=True)
        return carry

    lax.fori_loop(0, NBLK // 2, pair_a, 0)
    plsc.subcore_barrier()
    pltpu.sync_copy(den_sp.at[pl.ds(tile * NPT, NPT)],
                    den2.at[core].at[pl.ds(tile * NPT, NPT)])

    stages = (stage0, stage1)
    sems = (sem0, sem1)
    for h in range(8):
        plsc.subcore_barrier()

        def zrow(r, carry):
            rvec = jnp.full((16,), r, jnp.int32)
            for cc in range(c // 16):
                plsc.store_scatter(stage0, [rvec, cc * 16 + iota16], zeros16)
            return carry

        lax.fori_loop(0, KB, zrow, 0)
        for i in range(NPT // 128):
            pltpu.sync_copy(stage0, agg_sp.at[pl.ds(tile * NPT + i * 128, 128)])
        plsc.subcore_barrier()

        pltpu.async_copy(xps[h].at[s_idx.at[0]], stage0, sem0)

        def pair_b(p, carry):
            for b in range(2):
                blk = 2 * p + b
                stg = stages[b]
                pltpu.make_async_copy(xps[h].at[s_idx.at[blk]], stg, sems[b]).wait()

                @pl.when(blk >= 1)
                def _():
                    pltpu.make_async_copy(
                        stages[1 - b], agg_sp.at[d_idx.at[blk]],
                        ssems[1 - b]).wait()

                @pl.when(blk + 1 < NBLK)
                def _():
                    pltpu.async_copy(xps[h].at[s_idx.at[blk + 1]],
                                     stages[1 - b], sems[1 - b])

                def scale16(j16, carry2):
                    base = jnp.full((16,), blk * KB, jnp.int32)
                    for jj in range(8):
                        j = j16 * 8 + jj
                        jvec = jnp.full((16,), j, jnp.int32)
                        eev = plsc.load_gather(eetts[h], [base + jvec])
                        for cc in range(c // 16):
                            ci = cc * 16 + iota16
                            v = plsc.load_gather(stg, [jvec, ci])
                            plsc.store_scatter(stg, [jvec, ci], v * eev)
                    return carry2

                lax.fori_loop(0, KB // 8, scale16, 0)
                pltpu.async_copy(stg, agg_sp.at[d_idx.at[blk]], ssems[b],
                                 add=True)
            return carry

        lax.fori_loop(0, NBLK // 2, pair_b, 0)
        pltpu.make_async_copy(stage1, agg_sp.at[d_idx.at[0]],
                              ssems[(NBLK - 1) % 2]).wait()
        plsc.subcore_barrier()
        pltpu.sync_copy(agg_sp.at[pl.ds(tile * NPT, NPT)],
                        aggs[h].at[core].at[pl.ds(tile * NPT, NPT)])


def _gat_edge_sc(src2, dst2, asd, xps, c):
    mesh = plsc.VectorSubcoreMesh(core_axis_name="c", subcore_axis_name="s")
    out_type = ([jax.ShapeDtypeStruct((2, NP, 16), jnp.float32)] +
                [jax.ShapeDtypeStruct((2, NP, c), jnp.float32) for _ in range(8)])
    scratch = [
        pltpu.VMEM((NBLK, KB), jnp.int32),
        pltpu.VMEM((NBLK, KB), jnp.int32),
        pltpu.VMEM((KB, 16), jnp.float32),
        pltpu.VMEM((KB, 16), jnp.float32),
    ] + [pltpu.VMEM((EPW,), jnp.float32) for _ in range(8)] + [
        pltpu.VMEM((KB, c), jnp.float32),
        pltpu.VMEM((KB, c), jnp.float32),
        pltpu.VMEM((KB, 16), jnp.float32),
        pltpu.VMEM((KB, 16), jnp.float32),
        pltpu.VMEM((KB, 16), jnp.float32),
        pltpu.VMEM_SHARED((NP, 16), jnp.float32),
        pltpu.VMEM_SHARED((NP, c), jnp.float32),
        pltpu.SemaphoreType.DMA,
        pltpu.SemaphoreType.DMA,
        pltpu.SemaphoreType.DMA,
        pltpu.SemaphoreType.DMA,
    ]
    outs = pl.kernel(
        functools.partial(_edge_body, c),
        out_type=out_type,
        mesh=mesh,
        scratch_types=scratch,
        compiler_params=pltpu.CompilerParams(needs_layout_passes=False,
                                             use_tc_tiling_on_sc=False),
    )(src2, dst2, asd, *xps)
    return outs[0], outs[1:]


def _mm_body(x_ref, w_ref, o_ref):
    o_ref[...] = jnp.dot(x_ref[...], w_ref[...],
                         preferred_element_type=jnp.float32)


def _matmul(x, w, bm=400):
    m, k = x.shape
    n = w.shape[1]
    return pl.pallas_call(
        _mm_body,
        grid=(m // bm,),
        in_specs=[pl.BlockSpec((bm, k), lambda i: (i, 0)),
                  pl.BlockSpec((k, n), lambda i: (0, 0))],
        out_specs=pl.BlockSpec((bm, n), lambda i: (i, 0)),
        out_shape=jax.ShapeDtypeStruct((m, n), jnp.float32),
    )(x, w)


def _epi_body(c, with_res, *refs):
    if with_res:
        (den_ref, asd_ref, rpre_ref, b_ref, g_ref, bb_ref,
         rb_ref, rg_ref, rbb_ref, *rest) = refs
    else:
        (den_ref, asd_ref, b_ref, g_ref, bb_ref, *rest) = refs
    aggs = rest[0:8]
    xps = rest[8:16]
    o_ref = rest[16]
    cols = []
    for h in range(8):
        aggr = aggs[h][0] + aggs[h][1]
        xp = xps[h][...]
        d = den_ref[0, :, h:h + 1] + den_ref[1, :, h:h + 1]
        s = asd_ref[:, h:h + 1] + asd_ref[:, 8 + h:9 + h]
        eel = jnp.exp(jnp.where(s >= 0.0, s, 0.2 * s))
        cols.append((aggr + xp * eel) / (d + eel))
    x = jnp.concatenate(cols, axis=1) + b_ref[...]
    x = jnp.maximum(x, 0.0)
    x = x * (g_ref[...] * _BN_INV) + bb_ref[...]
    if with_res:
        r = (rpre_ref[...] + rb_ref[...]) * (rg_ref[...] * _BN_INV) + rbb_ref[...]
        x = x + r
    o_ref[...] = x


def _epilogue(den2, asd, aggs, xps, b, g, bb, c, rpre=None, rb=None, rg=None,
              rbb=None, bm=400):
    with_res = rpre is not None
    hc = H * c
    vec = lambda v: v.reshape(1, -1)
    args = [den2, asd]
    in_specs = [pl.BlockSpec((2, bm, 16), lambda i: (0, i, 0)),
                pl.BlockSpec((bm, 16), lambda i: (i, 0))]
    if with_res:
        args += [rpre]
        in_specs += [pl.BlockSpec((bm, hc), lambda i: (i, 0))]
    args += [vec(b), vec(g), vec(bb)]
    in_specs += [pl.BlockSpec((1, hc), lambda i: (0, 0))] * 3
    if with_res:
        args += [vec(rb), vec(rg), vec(rbb)]
        in_specs += [pl.BlockSpec((1, hc), lambda i: (0, 0))] * 3
    args += list(aggs) + list(xps)
    in_specs += [pl.BlockSpec((2, bm, c), lambda i: (0, i, 0))] * 8
    in_specs += [pl.BlockSpec((bm, c), lambda i: (i, 0))] * 8
    return pl.pallas_call(
        functools.partial(_epi_body, c, with_res),
        grid=(N // bm,),
        in_specs=in_specs,
        out_specs=pl.BlockSpec((bm, hc), lambda i: (i, 0)),
        out_shape=jax.ShapeDtypeStruct((N, hc), jnp.float32),
    )(*args)


def _pool_body(b_ref, h_ref, o_ref):
    @pl.when(pl.program_id(0) == 0)
    def _():
        o_ref[...] = jnp.zeros_like(o_ref)

    bi = b_ref[...]
    oh = (bi.T == lax.broadcasted_iota(jnp.int32, (NG, bi.shape[0]), 0))
    o_ref[...] += jnp.dot(oh.astype(jnp.float32), h_ref[...],
                          preferred_element_type=jnp.float32)


def _pool(batch, h4, bm=400):
    return pl.pallas_call(
        _pool_body,
        grid=(N // bm,),
        in_specs=[pl.BlockSpec((bm, 1), lambda i: (i, 0)),
                  pl.BlockSpec((bm, h4.shape[1]), lambda i: (i, 0))],
        out_specs=pl.BlockSpec((NG, h4.shape[1]), lambda i: (0, 0)),
        out_shape=jax.ShapeDtypeStruct((NG, h4.shape[1]), jnp.float32),
    )(batch.reshape(N, 1), h4)


def _fc_head_kernel(x_ref, *refs):
    (w1, b1, w2, b2, w3, b3, w4, b4,
     g1, bb1, g2, bb2, g3, bb3,
     rw1, rb1, rg1, rbb1, rw2, rb2, rg2, rbb2, rw3, rb3, rg3, rbb3,
     out_ref) = refs
    x = x_ref[...]
    ws = (w1, w2, w3)
    bs = (b1, b2, b3)
    gs = (g1, g2, g3)
    bbs = (bb1, bb2, bb3)
    rws = (rw1, rw2, rw3)
    rbs = (rb1, rb2, rb3)
    rgs = (rg1, rg2, rg3)
    rbbs = (rbb1, rbb2, rbb3)
    mm = lambda a, b: jnp.dot(a, b, preferred_element_type=jnp.float32)
    for i in range(3):
        r = (mm(x, rws[i][...]) + rbs[i][...]) * (rgs[i][...] * _BN_INV) + rbbs[i][...]
        y = jnp.maximum(mm(x, ws[i][...]) + bs[i][...], 0.0)
        x = y * (gs[i][...] * _BN_INV) + bbs[i][...] + r
    out_ref[...] = mm(x, w4[...]) + b4[...]


def _fc_head(p, x):
    args = [x, p['fc1_W'], p['fc1_b'], p['fc2_W'], p['fc2_b'], p['fc3_W'],
            p['fc3_b'], p['fc4_W'], p['fc4_b']]
    for i in (1, 2, 3):
        args += [p['fcbn%d_g' % i], p['fcbn%d_b' % i]]
    for i in (1, 2, 3):
        args += [p['fcres%d_W' % i], p['fcres%d_b' % i], p['fcres%d_g' % i],
                 p['fcres%d_bb' % i]]
    return pl.pallas_call(
        _fc_head_kernel,
        out_shape=jax.ShapeDtypeStruct((x.shape[0], 1), jnp.float32),
    )(*args)


def _head_mixers(asrc, adst, c):
    eye = jnp.eye(H, dtype=jnp.float32)
    a_s = (asrc[:, :, None] * eye[:, None, :]).reshape(H * c, H)
    a_d = (adst[:, :, None] * eye[:, None, :]).reshape(H * c, H)
    return jnp.concatenate([a_s, a_d], axis=1)


def _gat_layer(p, pre, x, src2, dst2, c, res_pre=None):
    W = p[pre + '_W']
    acat = _head_mixers(p[pre + '_asrc'], p[pre + '_adst'], c)
    parts = [W, W @ acat]
    if res_pre is not None:
        parts.append(p[res_pre + '_W'])
    wcat = jnp.concatenate(parts, axis=1)
    xpad = jnp.concatenate(
        [x, jnp.zeros((NP - N, x.shape[1]), jnp.float32)], axis=0)
    out = _matmul(xpad, wcat, bm=512)
    hc = H * c
    asd = out[:, hc:hc + 16]
    xps = [out[:, h * c:(h + 1) * c] for h in range(H)]
    rpre = out[:, hc + 16:] if res_pre is not None else None
    den2, aggs = _gat_edge_sc(src2, dst2, asd, xps, c)
    return den2, asd, aggs, xps, rpre


def _branch(p, g, x, ei, gfeat, batch):
    pad = EP - E
    src2 = jnp.concatenate([ei[0], jnp.zeros((pad,), ei.dtype)]).reshape(EP // KB, KB)
    dst2 = jnp.concatenate([ei[1], jnp.zeros((pad,), ei.dtype)]).reshape(EP // KB, KB)
    h = x
    for i in (1, 2, 3):
        pre = '%s_conv%d' % (g, i)
        rp = '%s_res%d' % (g, i)
        den2, asd, aggs, xps, rpre = _gat_layer(p, pre, h, src2, dst2, 64, rp)
        h = _epilogue(den2, asd, aggs, xps,
                      p[pre + '_b'], p['%s_bn%d_g' % (g, i)],
                      p['%s_bn%d_b' % (g, i)], 64,
                      rpre=rpre, rb=p[rp + '_b'], rg=p[rp + '_g'],
                      rbb=p[rp + '_bb'])
    pre = g + '_conv4'
    den2, asd, aggs, xps, _ = _gat_layer(p, pre, h, src2, dst2, 32)
    h4 = _epilogue(den2, asd, aggs, xps, p[pre + '_b'], p[g + '_bn4_g'],
                   p[g + '_bn4_b'], 32)
    pooled = _pool(batch, h4)
    return jnp.concatenate([pooled, gfeat], axis=1)


def kernel(params, solute_x, solute_edge_index, solute_edge_attr,
           solute_global_features, solute_batch, solvent_x, solvent_edge_index,
           solvent_edge_attr, solvent_global_features, solvent_batch):
    del solute_edge_attr, solvent_edge_attr
    p = params
    x1 = _branch(p, 's1', solute_x, solute_edge_index, solute_global_features,
                 solute_batch)
    x2 = _branch(p, 's2', solvent_x, solvent_edge_index,
                 solvent_global_features, solvent_batch)
    x = jnp.concatenate([x1, x2], axis=1)
    return _fc_head(p, x)
